# Initial kernel scaffold; baseline (speedup 1.0000x reference)
import jax
import jax.numpy as jnp
from jax.experimental import pallas as pl


def kernel(x, sl1_w, sl1_b, fc1_w, fc1_b, sl2_w, sl2_b, fc2_w, fc2_b, fc3_w, fc3_b, ro_w, ro_b, rows1, cols1, rows2, cols2):
    raise NotImplementedError("write your pallas kernel here")


# speedup vs baseline: 11.3982x; 11.3982x over previous
import jax
import jax.numpy as jnp
from jax.experimental import pallas as pl

_L1 = 128
_L2 = 64
_L3 = 32
_BM = 2048


def _dense_from_coo(w_ref, rows_ref, cols_ref, in_dim, out_pad):
    k = w_ref.shape[1]
    c_iota = jax.lax.broadcasted_iota(jnp.int32, (in_dim, k), 0)
    cw = jnp.where(cols_ref[0, :][None, :] == c_iota, w_ref[0, :][None, :], 0.0)
    r_iota = jax.lax.broadcasted_iota(jnp.int32, (out_pad, k), 0)
    r1h = jnp.where(rows_ref[0, :][None, :] == r_iota, 1.0, 0.0)
    return jax.lax.dot_general(
        cw, r1h, (((1,), (1,)), ((), ())),
        preferred_element_type=jnp.float32,
        precision=jax.lax.Precision.HIGHEST,
    )


def _hnn_block(x_ref, w1_ref, rows1_ref, cols1_ref, b1_ref, w2_ref, rows2_ref,
               cols2_ref, b2_ref, w3_ref, rows3_ref, cols3_ref, b3_ref,
               ro_ref, o_ref):
    hi = jax.lax.Precision.HIGHEST
    m1 = _dense_from_coo(w1_ref, rows1_ref, cols1_ref, _L1, 128)
    m2 = _dense_from_coo(w2_ref, rows2_ref, cols2_ref, _L2, 128)
    m3 = _dense_from_coo(w3_ref, rows3_ref, cols3_ref, _L3, 128)
    x = x_ref[...]
    t1 = jnp.maximum(
        jax.lax.dot_general(x, m1, (((1,), (0,)), ((), ())),
                            preferred_element_type=jnp.float32, precision=hi)
        + b1_ref[0, :][None, :], 0.0)
    s1 = t1[:, :_L2]
    t2 = jnp.maximum(
        jax.lax.dot_general(s1, m2, (((1,), (0,)), ((), ())),
                            preferred_element_type=jnp.float32, precision=hi)
        + b2_ref[0, :][None, :], 0.0)
    s2 = t2[:, :_L3]
    t3 = jnp.maximum(
        jax.lax.dot_general(s2, m3, (((1,), (0,)), ((), ())),
                            preferred_element_type=jnp.float32, precision=hi)
        + b3_ref[0, :][None, :], 0.0)
    o_ref[...] = (t1[:, _L2:_L2 + 1] * ro_ref[0, 0]
                  + t2[:, _L3:_L3 + 1] * ro_ref[0, 1]
                  + t3[:, 0:1] * ro_ref[0, 2]
                  + ro_ref[0, 3])


def kernel(x, sl1_w, sl1_b, fc1_w, fc1_b, sl2_w, sl2_b, fc2_w, fc2_b, fc3_w,
           fc3_b, ro_w, ro_b, rows1, cols1, rows2, cols2):
    b = x.shape[0]

    w1 = jnp.concatenate([sl1_w, fc1_w[0]]).reshape(1, -1)
    r1 = jnp.concatenate([rows1, jnp.full((_L1,), _L2, jnp.int32)]).reshape(1, -1)
    c1 = jnp.concatenate([cols1, jnp.arange(_L1, dtype=jnp.int32)]).reshape(1, -1)
    b1 = jnp.concatenate([sl1_b, fc1_b, jnp.zeros((128 - _L2 - 1,), jnp.float32)]).reshape(1, -1)
    w2 = jnp.concatenate([sl2_w, fc2_w[0]]).reshape(1, -1)
    r2 = jnp.concatenate([rows2, jnp.full((_L2,), _L3, jnp.int32)]).reshape(1, -1)
    c2 = jnp.concatenate([cols2, jnp.arange(_L2, dtype=jnp.int32)]).reshape(1, -1)
    b2 = jnp.concatenate([sl2_b, fc2_b, jnp.zeros((128 - _L3 - 1,), jnp.float32)]).reshape(1, -1)
    w3 = fc3_w[0].reshape(1, -1)
    r3 = jnp.zeros((1, _L3), jnp.int32)
    c3 = jnp.arange(_L3, dtype=jnp.int32).reshape(1, -1)
    b3 = jnp.concatenate([fc3_b, jnp.zeros((127,), jnp.float32)]).reshape(1, -1)
    ro = jnp.concatenate([ro_w[0], ro_b]).reshape(1, -1)

    small = lambda shp: pl.BlockSpec(shp, lambda i: (0, 0))
    return pl.pallas_call(
        _hnn_block,
        grid=(b // _BM,),
        in_specs=[
            pl.BlockSpec((_BM, _L1), lambda i: (i, 0)),
            small((1, _L1 + _L1)), small((1, _L1 + _L1)), small((1, _L1 + _L1)),
            small((1, 128)),
            small((1, _L2 + _L2)), small((1, _L2 + _L2)), small((1, _L2 + _L2)),
            small((1, 128)),
            small((1, _L3)), small((1, _L3)), small((1, _L3)),
            small((1, 128)),
            small((1, 4)),
        ],
        out_specs=pl.BlockSpec((_BM, 1), lambda i: (i, 0)),
        out_shape=jax.ShapeDtypeStruct((b, 1), jnp.float32),
    )(x, w1, r1, c1, b1, w2, r2, c2, b2, w3, r3, c3, b3, ro)
